# baseline (speedup 1.0000x reference)
import functools

import jax
import jax.numpy as jnp
from jax import lax
from jax.experimental import pallas as pl
from jax.experimental.pallas import tpu as pltpu
from jax.experimental.pallas import tpu_sc as plsc

DATA_DIM = 1000000
D = 32
BATCH = 16384
HIST = 50
N = BATCH * HIST
NUM_WORKERS = 32
BLK = 128
NUM_BLOCKS = N // BLK
BLOCKS_PER_W = NUM_BLOCKS // NUM_WORKERS
RROWS = DATA_DIM // 4
NCHUNK_FULL = DATA_DIM // BLK
TAIL_COLS = DATA_DIM - NCHUNK_FULL * BLK

_mesh = plsc.VectorSubcoreMesh(core_axis_name="c", subcore_axis_name="s")


@functools.partial(
    pl.kernel,
    mesh=_mesh,
    out_type=(
        jax.ShapeDtypeStruct((RROWS, 128), jnp.float32),
        jax.ShapeDtypeStruct((N,), jnp.int32),
    ),
    scratch_types=[
        pltpu.VMEM((D, BLK), jnp.float32),
        pltpu.VMEM((D, BLK), jnp.float32),
        pltpu.VMEM((BATCH,), jnp.int32),
    ],
    compiler_params=pltpu.CompilerParams(needs_layout_passes=False),
)
def _reformat(xt_hbm, tt_hbm, r_hbm, xlin_hbm, in_v, out_v, row_v):
    wid = lax.axis_index("s") * 2 + lax.axis_index("c")

    def shuffle(nrows):
        it = jax.lax.broadcasted_iota(jnp.int32, (16,), 0)
        for r in range(nrows):
            for t in range(8):
                rows = it + 16 * (t & 1)
                cols = jnp.full((16,), 4 * r + t // 2, jnp.int32)
                out_v[r, pl.ds(t * 16, 16)] = plsc.load_gather(in_v, [rows, cols])

    def chunk_body(g, _):
        c = wid + g * NUM_WORKERS
        pltpu.sync_copy(tt_hbm.at[:, pl.ds(c * BLK, BLK)], in_v)
        shuffle(32)
        pltpu.sync_copy(out_v, r_hbm.at[pl.ds(c * D, D), :])
        return 0

    ntrips = (NCHUNK_FULL - 1 - wid) // NUM_WORKERS + 1
    lax.fori_loop(0, ntrips, chunk_body, 0, unroll=False)

    @pl.when(wid == NUM_WORKERS - 1)
    def _():
        start = pl.multiple_of(NCHUNK_FULL * BLK + (wid - (NUM_WORKERS - 1)), BLK)
        pltpu.sync_copy(tt_hbm.at[:, pl.ds(start, BLK)], in_v)
        shuffle(TAIL_COLS // 4)
        pltpu.sync_copy(
            out_v.at[pl.ds(0, TAIL_COLS // 4), :],
            r_hbm.at[pl.ds(NCHUNK_FULL * D, TAIL_COLS // 4), :],
        )

    def xrow_body(g, _):
        h = wid + g * NUM_WORKERS
        pltpu.sync_copy(xt_hbm.at[h], row_v)
        pltpu.sync_copy(row_v, xlin_hbm.at[pl.ds(h * BATCH, BATCH)])
        return 0

    xtrips = (HIST - 1 - wid) // NUM_WORKERS + 1
    lax.fori_loop(0, xtrips, xrow_body, 0, unroll=False)


@functools.partial(
    pl.kernel,
    mesh=_mesh,
    out_type=jax.ShapeDtypeStruct((HIST, 4, BATCH // BLK, 8, BLK), jnp.float32),
    scratch_types=[
        pltpu.VMEM((2, BLK), jnp.int32),
        pltpu.VMEM((2, BLK, D), jnp.float32),
        pltpu.VMEM((4, 8, BLK), jnp.float32),
        pltpu.SemaphoreType.DMA,
        pltpu.SemaphoreType.DMA,
    ],
    compiler_params=pltpu.CompilerParams(
        needs_layout_passes=False, use_tc_tiling_on_sc=False
    ),
)
def _gather_all(xlin_hbm, rl_hbm, out_hbm, i_v, rows_v, stage_v, sem0, sem1):
    wid = lax.axis_index("s") * 2 + lax.axis_index("c")
    k0 = wid * BLOCKS_PER_W
    sems = (sem0, sem1)

    def issue(k, buf):
        h = k // 128
        c = k % 128
        pltpu.sync_copy(xlin_hbm.at[pl.ds(h * BATCH + c * BLK, BLK)], i_v.at[buf])
        pltpu.async_copy(rl_hbm.at[i_v.at[buf]], rows_v.at[buf], sems[buf])

    def drain(k, buf):
        h = k // 128
        c = k % 128
        pltpu.make_async_copy(
            rl_hbm.at[i_v.at[buf]], rows_v.at[buf], sems[buf]
        ).wait()

        def extract(t, _):
            rt = jax.lax.broadcasted_iota(jnp.int32, (16,), 0) + t * 16
            for d in range(D):
                cd = jnp.full((16,), d, jnp.int32)
                stage_v[d // 8, d % 8, pl.ds(t * 16, 16)] = plsc.load_gather(
                    rows_v.at[buf], [rt, cd]
                )
            return 0

        lax.fori_loop(0, BLK // 16, extract, 0, unroll=False)
        pltpu.sync_copy(stage_v, out_hbm.at[h, :, c, :, :])

    issue(k0, 0)

    def body(g, _):
        k = k0 + g * 2
        issue(k + 1, 1)
        drain(k, 0)
        issue(k + 2, 0)
        drain(k + 1, 1)
        return 0

    lax.fori_loop(0, (BLOCKS_PER_W - 2) // 2, body, 0, unroll=False)
    issue(k0 + BLOCKS_PER_W - 1, 1)
    drain(k0 + BLOCKS_PER_W - 2, 0)
    drain(k0 + BLOCKS_PER_W - 1, 1)


def kernel(x, table):
    r, xlin = _reformat(x.T, table.T)
    rl = r.reshape(DATA_DIM, D)
    out5 = _gather_all(xlin, rl)
    out = out5.transpose(0, 1, 3, 2, 4).reshape(HIST, D, BATCH)
    return out.transpose(2, 0, 1)


# speedup vs baseline: 1.2466x; 1.2466x over previous
import functools

import jax
import jax.numpy as jnp
from jax import lax
from jax.experimental import pallas as pl
from jax.experimental.pallas import tpu as pltpu
from jax.experimental.pallas import tpu_sc as plsc

DATA_DIM = 1000000
D = 32
BATCH = 16384
HIST = 50
N = BATCH * HIST
NUM_WORKERS = 32
RROWS = DATA_DIM // 4
GW = 512
NG_EVEN = 1952
G_PER_W = NG_EVEN // NUM_WORKERS
TAIL_COLS = 64
STRIPE = 512

_mesh = plsc.VectorSubcoreMesh(core_axis_name="c", subcore_axis_name="s")


@functools.partial(
    pl.kernel,
    mesh=_mesh,
    out_type=(
        jax.ShapeDtypeStruct((RROWS, 128), jnp.float32),
        jax.ShapeDtypeStruct((N,), jnp.int32),
    ),
    scratch_types=[
        pltpu.VMEM((2, D, GW), jnp.float32),
        pltpu.VMEM((GW // 4, 128), jnp.float32),
        pltpu.VMEM((D, 128), jnp.float32),
        pltpu.VMEM((BATCH,), jnp.int32),
        pltpu.SemaphoreType.DMA,
        pltpu.SemaphoreType.DMA,
    ],
    compiler_params=pltpu.CompilerParams(needs_layout_passes=False),
)
def _reformat(xt_hbm, tt_hbm, r_hbm, xlin_hbm, in_v, out_v, tail_v, row_v, si0, si1):
    wid = lax.axis_index("s") * 2 + lax.axis_index("c")
    g0 = wid * G_PER_W
    sems = (si0, si1)
    it = jax.lax.broadcasted_iota(jnp.int32, (16,), 0)

    def start_in(g, buf):
        pltpu.async_copy(tt_hbm.at[:, pl.ds(g * GW, GW)], in_v.at[buf], sems[buf])

    def wait_in(g, buf):
        pltpu.make_async_copy(
            tt_hbm.at[:, pl.ds(g * GW, GW)], in_v.at[buf], sems[buf]
        ).wait()

    def shuffle_store(g, buf):
        def srow(r, _):
            for t in range(8):
                rows = it + 16 * (t & 1)
                cols = jnp.full((16,), 4 * r + t // 2, jnp.int32)
                out_v[r, pl.ds(t * 16, 16)] = plsc.load_gather(
                    in_v.at[buf], [rows, cols]
                )
            return 0

        lax.fori_loop(0, GW // 4, srow, 0, unroll=False)
        pltpu.sync_copy(out_v, r_hbm.at[pl.ds(g * (GW // 4), GW // 4), :])

    start_in(g0, 0)

    def body(i, _):
        ga = g0 + 2 * i
        start_in(ga + 1, 1)
        wait_in(ga, 0)
        shuffle_store(ga, 0)

        @pl.when(2 * i + 2 < G_PER_W)
        def _():
            start_in(ga + 2, 0)

        wait_in(ga + 1, 1)
        shuffle_store(ga + 1, 1)
        return 0

    lax.fori_loop(0, (G_PER_W - 1) // 2, body, 0, unroll=False)
    glast = g0 + G_PER_W - 1
    wait_in(glast, 0)
    shuffle_store(glast, 0)

    @pl.when(wid == NUM_WORKERS - 2)
    def _():
        start_in(NG_EVEN, 0)
        wait_in(NG_EVEN, 0)
        shuffle_store(NG_EVEN, 0)

    @pl.when(wid == NUM_WORKERS - 1)
    def _():
        start = pl.multiple_of((NG_EVEN + 1) * GW + (wid - (NUM_WORKERS - 1)), 128)
        pltpu.sync_copy(tt_hbm.at[:, pl.ds(start, 128)], tail_v)

        def trow(r, _):
            for t in range(8):
                rows = it + 16 * (t & 1)
                cols = jnp.full((16,), 4 * r + t // 2, jnp.int32)
                out_v[r, pl.ds(t * 16, 16)] = plsc.load_gather(tail_v, [rows, cols])
            return 0

        lax.fori_loop(0, TAIL_COLS // 4, trow, 0, unroll=False)
        pltpu.sync_copy(
            out_v.at[pl.ds(0, TAIL_COLS // 4), :],
            r_hbm.at[pl.ds((NG_EVEN + 1) * (GW // 4), TAIL_COLS // 4), :],
        )

    def xrow_body(i, _):
        h = wid + i * NUM_WORKERS
        pltpu.sync_copy(xt_hbm.at[h], row_v)
        pltpu.sync_copy(row_v, xlin_hbm.at[pl.ds(h * BATCH, BATCH)])
        return 0

    xtrips = (HIST - 1 - wid) // NUM_WORKERS + 1
    lax.fori_loop(0, xtrips, xrow_body, 0, unroll=False)


@functools.partial(
    pl.kernel,
    mesh=_mesh,
    out_type=jax.ShapeDtypeStruct((HIST, 4, BATCH // 128, 8, 128), jnp.float32),
    scratch_types=[
        pltpu.VMEM((HIST, STRIPE), jnp.int32),
        pltpu.VMEM((2, STRIPE, D), jnp.float32),
        pltpu.VMEM((4, 4, 8, 128), jnp.float32),
        pltpu.SemaphoreType.DMA,
        pltpu.SemaphoreType.DMA,
    ],
    compiler_params=pltpu.CompilerParams(
        needs_layout_passes=False, use_tc_tiling_on_sc=False
    ),
)
def _gather_all(x2_hbm, rl_hbm, out_hbm, i_v, rows_v, stage_v, sem0, sem1):
    wid = lax.axis_index("s") * 2 + lax.axis_index("c")
    c0 = wid * 4
    sems = (sem0, sem1)
    it = jax.lax.broadcasted_iota(jnp.int32, (16,), 0)

    pltpu.sync_copy(x2_hbm.at[:, pl.ds(c0 * 128, STRIPE)], i_v)

    def issue(h, buf):
        pltpu.async_copy(rl_hbm.at[i_v.at[h]], rows_v.at[buf], sems[buf])

    def drain(h, buf):
        pltpu.make_async_copy(
            rl_hbm.at[i_v.at[h]], rows_v.at[buf], sems[buf]
        ).wait()
        for ci in range(4):
            def extract(t, _):
                rt = it + (128 * ci + 16 * t)
                for d in range(D):
                    cd = jnp.full((16,), d, jnp.int32)
                    stage_v[d // 8, ci, d % 8, pl.ds(t * 16, 16)] = plsc.load_gather(
                        rows_v.at[buf], [rt, cd]
                    )
                return 0

            lax.fori_loop(0, 8, extract, 0, unroll=False)
        pltpu.sync_copy(stage_v, out_hbm.at[h, :, pl.ds(c0, 4), :, :])

    issue(0, 0)

    def body(g, _):
        h = 2 * g
        issue(h + 1, 1)
        drain(h, 0)
        issue(h + 2, 0)
        drain(h + 1, 1)
        return 0

    lax.fori_loop(0, (HIST - 2) // 2, body, 0, unroll=False)
    issue(HIST - 1, 1)
    drain(HIST - 2, 0)
    drain(HIST - 1, 1)


def kernel(x, table):
    r, xlin = _reformat(x.T, table.T)
    rl = r.reshape(DATA_DIM, D)
    x2 = xlin.reshape(HIST, BATCH)
    out5 = _gather_all(x2, rl)
    out = out5.transpose(0, 1, 3, 2, 4).reshape(HIST, D, BATCH)
    return out.transpose(2, 0, 1)


# speedup vs baseline: 3.3528x; 2.6895x over previous
import functools

import jax
import jax.numpy as jnp
from jax import lax
from jax.experimental import pallas as pl
from jax.experimental.pallas import tpu as pltpu
from jax.experimental.pallas import tpu_sc as plsc

DATA_DIM = 1000000
D = 32
BATCH = 16384
HIST = 50
N = BATCH * HIST
NUM_WORKERS = 32
RROWS = DATA_DIM // 4
GW = 512
NG_EVEN = 1952
G_PER_W = NG_EVEN // NUM_WORKERS
TAIL_COLS = 64
STRIPE = 512

_mesh = plsc.VectorSubcoreMesh(core_axis_name="c", subcore_axis_name="s")


@functools.partial(
    pl.kernel,
    mesh=_mesh,
    out_type=(
        jax.ShapeDtypeStruct((RROWS, 128), jnp.float32),
        jax.ShapeDtypeStruct((N,), jnp.int32),
    ),
    scratch_types=[
        pltpu.VMEM((2, D, GW), jnp.float32),
        pltpu.VMEM((2, GW // 4, 128), jnp.float32),
        pltpu.VMEM((D, 128), jnp.float32),
        pltpu.VMEM((BATCH,), jnp.int32),
        pltpu.SemaphoreType.DMA,
        pltpu.SemaphoreType.DMA,
        pltpu.SemaphoreType.DMA,
        pltpu.SemaphoreType.DMA,
    ],
    compiler_params=pltpu.CompilerParams(needs_layout_passes=False),
)
def _reformat(
    xt_hbm, tt_hbm, r_hbm, xlin_hbm, in_v, out_v, tail_v, row_v, si0, si1, so0, so1
):
    wid = lax.axis_index("s") * 2 + lax.axis_index("c")
    g0 = wid * G_PER_W
    isems = (si0, si1)
    osems = (so0, so1)
    it = jax.lax.broadcasted_iota(jnp.int32, (16,), 0)
    c_e4 = lax.shift_right_logical(it, 2)
    c_k32 = (it & 3) * D

    def start_in(g, buf):
        pltpu.async_copy(tt_hbm.at[:, pl.ds(g * GW, GW)], in_v.at[buf], isems[buf])

    def wait_in(g, buf):
        pltpu.make_async_copy(
            tt_hbm.at[:, pl.ds(g * GW, GW)], in_v.at[buf], isems[buf]
        ).wait()

    def start_out(g, buf):
        pltpu.async_copy(
            out_v.at[buf], r_hbm.at[pl.ds(g * (GW // 4), GW // 4), :], osems[buf]
        )

    def wait_out(buf):
        pltpu.make_async_copy(
            out_v.at[buf], r_hbm.at[pl.ds(0, GW // 4), :], osems[buf]
        ).wait()

    def shuffle(src, nrows, obuf):
        def sdiag(m, _):
            e0 = m * 16
            e_vec = e0 + it
            er_vec = m * 4 + c_e4
            for d0 in range(D):
                d_vec = (d0 + it) & (D - 1)
                v = plsc.load_gather(src, [d_vec, e_vec])
                plsc.store_scatter(out_v.at[obuf], [er_vec, c_k32 + d_vec], v)
            return 0

        lax.fori_loop(0, nrows * 4 // 16, sdiag, 0, unroll=False)

    start_in(g0, 0)

    def body(i, _):
        ga = g0 + 2 * i
        start_in(ga + 1, 1)
        wait_in(ga, 0)

        @pl.when(i > 0)
        def _():
            wait_out(0)

        shuffle(in_v.at[0], GW // 4, 0)
        start_out(ga, 0)

        @pl.when(2 * i + 2 < G_PER_W)
        def _():
            start_in(ga + 2, 0)

        wait_in(ga + 1, 1)

        @pl.when(i > 0)
        def _():
            wait_out(1)

        shuffle(in_v.at[1], GW // 4, 1)
        start_out(ga + 1, 1)
        return 0

    npairs = (G_PER_W - 1) // 2
    lax.fori_loop(0, npairs, body, 0, unroll=False)
    glast = g0 + G_PER_W - 1
    wait_in(glast, 0)
    wait_out(0)
    shuffle(in_v.at[0], GW // 4, 0)
    start_out(glast, 0)
    wait_out(1)
    wait_out(0)

    @pl.when(wid == NUM_WORKERS - 2)
    def _():
        start_in(NG_EVEN, 0)
        wait_in(NG_EVEN, 0)
        shuffle(in_v.at[0], GW // 4, 0)
        pltpu.sync_copy(
            out_v.at[0], r_hbm.at[pl.ds(NG_EVEN * (GW // 4), GW // 4), :]
        )

    @pl.when(wid == NUM_WORKERS - 1)
    def _():
        start = pl.multiple_of((NG_EVEN + 1) * GW + (wid - (NUM_WORKERS - 1)), 128)
        pltpu.sync_copy(tt_hbm.at[:, pl.ds(start, 128)], tail_v)
        shuffle(tail_v, TAIL_COLS // 4, 0)
        pltpu.sync_copy(
            out_v.at[0, pl.ds(0, TAIL_COLS // 4), :],
            r_hbm.at[pl.ds((NG_EVEN + 1) * (GW // 4), TAIL_COLS // 4), :],
        )

    def xrow_body(i, _):
        h = wid + i * NUM_WORKERS
        pltpu.sync_copy(xt_hbm.at[h], row_v)
        pltpu.sync_copy(row_v, xlin_hbm.at[pl.ds(h * BATCH, BATCH)])
        return 0

    xtrips = (HIST - 1 - wid) // NUM_WORKERS + 1
    lax.fori_loop(0, xtrips, xrow_body, 0, unroll=False)


@functools.partial(
    pl.kernel,
    mesh=_mesh,
    out_type=jax.ShapeDtypeStruct((HIST, 4, BATCH // 128, 8, 128), jnp.float32),
    scratch_types=[
        pltpu.VMEM((HIST, STRIPE), jnp.int32),
        pltpu.VMEM((2, STRIPE, D), jnp.float32),
        pltpu.VMEM((4, 4, 8, 128), jnp.float32),
        pltpu.SemaphoreType.DMA,
        pltpu.SemaphoreType.DMA,
    ],
    compiler_params=pltpu.CompilerParams(
        needs_layout_passes=False, use_tc_tiling_on_sc=False
    ),
)
def _gather_all(x2_hbm, rl_hbm, out_hbm, i_v, rows_v, stage_v, sem0, sem1):
    wid = lax.axis_index("s") * 2 + lax.axis_index("c")
    c0 = wid * 4
    sems = (sem0, sem1)
    it = jax.lax.broadcasted_iota(jnp.int32, (16,), 0)

    pltpu.sync_copy(x2_hbm.at[:, pl.ds(c0 * 128, STRIPE)], i_v)

    def issue(h, buf):
        pltpu.async_copy(rl_hbm.at[i_v.at[h]], rows_v.at[buf], sems[buf])

    def drain(h, buf):
        pltpu.make_async_copy(
            rl_hbm.at[i_v.at[h]], rows_v.at[buf], sems[buf]
        ).wait()
        for ci in range(4):

            def extract(m, _, _ci=ci):
                n0 = m * 16
                r_vec = (128 * _ci + n0) + it
                n_vec = n0 + it
                for d0 in range(D):
                    d_vec = (d0 + it) & (D - 1)
                    v = plsc.load_gather(rows_v.at[buf], [r_vec, d_vec])
                    plsc.store_scatter(
                        stage_v.at[_ci],
                        [lax.shift_right_logical(d_vec, 3), d_vec & 7, n_vec],
                        v,
                    )
                return 0

            lax.fori_loop(0, 8, extract, 0, unroll=False)
        for ci in range(4):
            pltpu.sync_copy(stage_v.at[ci], out_hbm.at[h, :, c0 + ci, :, :])

    issue(0, 0)

    def body(g, _):
        h = 2 * g
        issue(h + 1, 1)
        drain(h, 0)
        issue(h + 2, 0)
        drain(h + 1, 1)
        return 0

    lax.fori_loop(0, (HIST - 2) // 2, body, 0, unroll=False)
    issue(HIST - 1, 1)
    drain(HIST - 2, 0)
    drain(HIST - 1, 1)


def kernel(x, table):
    r, xlin = _reformat(x.T, table.T)
    rl = r.reshape(DATA_DIM, D)
    x2 = xlin.reshape(HIST, BATCH)
    out5 = _gather_all(x2, rl)
    out = out5.transpose(0, 1, 3, 2, 4).reshape(HIST, D, BATCH)
    return out.transpose(2, 0, 1)


# speedup vs baseline: 3.5971x; 1.0729x over previous
import functools

import jax
import jax.numpy as jnp
from jax import lax
from jax.experimental import pallas as pl
from jax.experimental.pallas import tpu as pltpu
from jax.experimental.pallas import tpu_sc as plsc

DATA_DIM = 1000000
D = 32
BATCH = 16384
HIST = 50
N = BATCH * HIST
NUM_WORKERS = 32
RROWS = DATA_DIM // 4
GW = 512
NG_EVEN = 1952
G_PER_W = NG_EVEN // NUM_WORKERS
TAIL_COLS = 64
STRIPE = 512

_mesh = plsc.VectorSubcoreMesh(core_axis_name="c", subcore_axis_name="s")


@functools.partial(
    pl.kernel,
    mesh=_mesh,
    out_type=(
        jax.ShapeDtypeStruct((RROWS, 128), jnp.float32),
        jax.ShapeDtypeStruct((N,), jnp.int32),
    ),
    scratch_types=[
        pltpu.VMEM((2, D, GW), jnp.float32),
        pltpu.VMEM((2, GW // 4, 128), jnp.float32),
        pltpu.VMEM((D, 128), jnp.float32),
        pltpu.VMEM((BATCH,), jnp.int32),
        pltpu.SemaphoreType.DMA,
        pltpu.SemaphoreType.DMA,
        pltpu.SemaphoreType.DMA,
        pltpu.SemaphoreType.DMA,
    ],
    compiler_params=pltpu.CompilerParams(needs_layout_passes=False),
)
def _reformat(
    xt_hbm, tt_hbm, r_hbm, xlin_hbm, in_v, out_v, tail_v, row_v, si0, si1, so0, so1
):
    wid = lax.axis_index("s") * 2 + lax.axis_index("c")
    g0 = wid * G_PER_W
    isems = (si0, si1)
    osems = (so0, so1)
    it = jax.lax.broadcasted_iota(jnp.int32, (16,), 0)
    c_e4 = lax.shift_right_logical(it, 2)
    c_k32 = (it & 3) * D

    def start_in(g, buf):
        pltpu.async_copy(tt_hbm.at[:, pl.ds(g * GW, GW)], in_v.at[buf], isems[buf])

    def wait_in(g, buf):
        pltpu.make_async_copy(
            tt_hbm.at[:, pl.ds(g * GW, GW)], in_v.at[buf], isems[buf]
        ).wait()

    def start_out(g, buf):
        pltpu.async_copy(
            out_v.at[buf], r_hbm.at[pl.ds(g * (GW // 4), GW // 4), :], osems[buf]
        )

    def wait_out(buf):
        pltpu.make_async_copy(
            out_v.at[buf], r_hbm.at[pl.ds(0, GW // 4), :], osems[buf]
        ).wait()

    def shuffle(src, nrows, obuf):
        def sdiag(m, _):
            e0 = m * 16
            e_vec = e0 + it
            er_vec = m * 4 + c_e4
            for d0 in range(D):
                d_vec = (d0 + it) & (D - 1)
                v = plsc.load_gather(src, [d_vec, e_vec])
                plsc.store_scatter(out_v.at[obuf], [er_vec, c_k32 + d_vec], v)
            return 0

        lax.fori_loop(0, nrows * 4 // 16, sdiag, 0, unroll=False)

    start_in(g0, 0)

    def body(i, _):
        ga = g0 + 2 * i
        start_in(ga + 1, 1)
        wait_in(ga, 0)

        @pl.when(i > 0)
        def _():
            wait_out(0)

        shuffle(in_v.at[0], GW // 4, 0)
        start_out(ga, 0)

        @pl.when(2 * i + 2 < G_PER_W)
        def _():
            start_in(ga + 2, 0)

        wait_in(ga + 1, 1)

        @pl.when(i > 0)
        def _():
            wait_out(1)

        shuffle(in_v.at[1], GW // 4, 1)
        start_out(ga + 1, 1)
        return 0

    npairs = (G_PER_W - 1) // 2
    lax.fori_loop(0, npairs, body, 0, unroll=False)
    glast = g0 + G_PER_W - 1
    wait_in(glast, 0)
    wait_out(0)
    shuffle(in_v.at[0], GW // 4, 0)
    start_out(glast, 0)
    wait_out(1)
    wait_out(0)

    @pl.when(wid == NUM_WORKERS - 2)
    def _():
        start_in(NG_EVEN, 0)
        wait_in(NG_EVEN, 0)
        shuffle(in_v.at[0], GW // 4, 0)
        pltpu.sync_copy(
            out_v.at[0], r_hbm.at[pl.ds(NG_EVEN * (GW // 4), GW // 4), :]
        )

    @pl.when(wid == NUM_WORKERS - 1)
    def _():
        start = pl.multiple_of((NG_EVEN + 1) * GW + (wid - (NUM_WORKERS - 1)), 128)
        pltpu.sync_copy(tt_hbm.at[:, pl.ds(start, 128)], tail_v)
        shuffle(tail_v, TAIL_COLS // 4, 0)
        pltpu.sync_copy(
            out_v.at[0, pl.ds(0, TAIL_COLS // 4), :],
            r_hbm.at[pl.ds((NG_EVEN + 1) * (GW // 4), TAIL_COLS // 4), :],
        )

    def xrow_body(i, _):
        h = wid + i * NUM_WORKERS
        pltpu.sync_copy(xt_hbm.at[h], row_v)
        pltpu.sync_copy(row_v, xlin_hbm.at[pl.ds(h * BATCH, BATCH)])
        return 0

    xtrips = (HIST - 1 - wid) // NUM_WORKERS + 1
    lax.fori_loop(0, xtrips, xrow_body, 0, unroll=False)


@functools.partial(
    pl.kernel,
    mesh=_mesh,
    out_type=jax.ShapeDtypeStruct((HIST, 4, BATCH // 128, 8, 128), jnp.float32),
    scratch_types=[
        pltpu.VMEM((HIST, STRIPE), jnp.int32),
        pltpu.VMEM((2, STRIPE, D), jnp.float32),
        pltpu.VMEM((2, 4, 4, 8, 128), jnp.float32),
        pltpu.SemaphoreType.DMA,
        pltpu.SemaphoreType.DMA,
        pltpu.SemaphoreType.DMA,
        pltpu.SemaphoreType.DMA,
    ],
    compiler_params=pltpu.CompilerParams(
        needs_layout_passes=False, use_tc_tiling_on_sc=False
    ),
)
def _gather_all(x2_hbm, rl_hbm, out_hbm, i_v, rows_v, stage_v, sem0, sem1, os0, os1):
    wid = lax.axis_index("s") * 2 + lax.axis_index("c")
    c0 = wid * 4
    sems = (sem0, sem1)
    osems = (os0, os1)
    it = jax.lax.broadcasted_iota(jnp.int32, (16,), 0)

    pltpu.sync_copy(x2_hbm.at[:, pl.ds(c0 * 128, STRIPE)], i_v)

    def issue(h, buf):
        pltpu.async_copy(rl_hbm.at[i_v.at[h]], rows_v.at[buf], sems[buf])

    def wait_stage(h, buf):
        for ci in range(4):
            pltpu.make_async_copy(
                stage_v.at[buf, ci], out_hbm.at[h, :, c0 + ci, :, :], osems[buf]
            ).wait()

    def drain(h, buf):
        pltpu.make_async_copy(
            rl_hbm.at[i_v.at[h]], rows_v.at[buf], sems[buf]
        ).wait()

        @pl.when(h >= 2)
        def _():
            wait_stage(h, buf)

        for ci in range(4):

            def extract(m, _, _ci=ci):
                n0 = m * 16
                r_vec = (128 * _ci + n0) + it
                n_vec = n0 + it
                for d0 in range(D):
                    d_vec = (d0 + it) & (D - 1)
                    v = plsc.load_gather(rows_v.at[buf], [r_vec, d_vec])
                    plsc.store_scatter(
                        stage_v.at[buf, _ci],
                        [lax.shift_right_logical(d_vec, 3), d_vec & 7, n_vec],
                        v,
                    )
                return 0

            lax.fori_loop(0, 8, extract, 0, unroll=False)
        for ci in range(4):
            pltpu.async_copy(
                stage_v.at[buf, ci], out_hbm.at[h, :, c0 + ci, :, :], osems[buf]
            )

    issue(0, 0)

    def body(g, _):
        h = 2 * g
        issue(h + 1, 1)
        drain(h, 0)
        issue(h + 2, 0)
        drain(h + 1, 1)
        return 0

    lax.fori_loop(0, (HIST - 2) // 2, body, 0, unroll=False)
    issue(HIST - 1, 1)
    drain(HIST - 2, 0)
    drain(HIST - 1, 1)
    wait_stage(HIST - 2, 0)
    wait_stage(HIST - 1, 1)


def kernel(x, table):
    r, xlin = _reformat(x.T, table.T)
    rl = r.reshape(DATA_DIM, D)
    x2 = xlin.reshape(HIST, BATCH)
    out5 = _gather_all(x2, rl)
    out = out5.transpose(0, 1, 3, 2, 4).reshape(HIST, D, BATCH)
    return out.transpose(2, 0, 1)


# speedup vs baseline: 3.6026x; 1.0015x over previous
import functools

import jax
import jax.numpy as jnp
from jax import lax
from jax.experimental import pallas as pl
from jax.experimental.pallas import tpu as pltpu
from jax.experimental.pallas import tpu_sc as plsc

DATA_DIM = 1000000
D = 32
BATCH = 16384
HIST = 50
N = BATCH * HIST
NUM_WORKERS = 32
RROWS = DATA_DIM // 4
GW = 512
NG_EVEN = 1952
G_PER_W = NG_EVEN // NUM_WORKERS
TAIL_COLS = 64
STRIPE = 512

_mesh = plsc.VectorSubcoreMesh(core_axis_name="c", subcore_axis_name="s")


@functools.partial(
    pl.kernel,
    mesh=_mesh,
    out_type=(
        jax.ShapeDtypeStruct((RROWS, 128), jnp.float32),
        jax.ShapeDtypeStruct((N,), jnp.int32),
    ),
    scratch_types=[
        pltpu.VMEM((2, D, GW), jnp.float32),
        pltpu.VMEM((2, GW // 4, 128), jnp.float32),
        pltpu.VMEM((D, 128), jnp.float32),
        pltpu.VMEM((BATCH,), jnp.int32),
        pltpu.SemaphoreType.DMA,
        pltpu.SemaphoreType.DMA,
        pltpu.SemaphoreType.DMA,
        pltpu.SemaphoreType.DMA,
    ],
    compiler_params=pltpu.CompilerParams(needs_layout_passes=False),
)
def _reformat(
    xt_hbm, tt_hbm, r_hbm, xlin_hbm, in_v, out_v, tail_v, row_v, si0, si1, so0, so1
):
    wid = lax.axis_index("s") * 2 + lax.axis_index("c")
    g0 = wid * G_PER_W
    isems = (si0, si1)
    osems = (so0, so1)
    it = jax.lax.broadcasted_iota(jnp.int32, (16,), 0)
    c_e4 = lax.shift_right_logical(it, 2)
    c_k32 = (it & 3) * D

    def start_in(g, buf):
        pltpu.async_copy(tt_hbm.at[:, pl.ds(g * GW, GW)], in_v.at[buf], isems[buf])

    def wait_in(g, buf):
        pltpu.make_async_copy(
            tt_hbm.at[:, pl.ds(g * GW, GW)], in_v.at[buf], isems[buf]
        ).wait()

    def start_out(g, buf):
        pltpu.async_copy(
            out_v.at[buf], r_hbm.at[pl.ds(g * (GW // 4), GW // 4), :], osems[buf]
        )

    def wait_out(buf):
        pltpu.make_async_copy(
            out_v.at[buf], r_hbm.at[pl.ds(0, GW // 4), :], osems[buf]
        ).wait()

    def shuffle(src, nrows, obuf):
        def sdiag(m, _):
            e_vec = m * 16 + it
            er_vec = m * 4 + c_e4
            for d0 in range(D):
                d_vec = d0 + it if d0 < 16 else (d0 + it) & (D - 1)
                v = plsc.load_gather(src, [d_vec, e_vec])
                plsc.store_scatter(out_v.at[obuf], [er_vec, c_k32 + d_vec], v)
            return 0

        lax.fori_loop(0, nrows * 4 // 16, sdiag, 0, unroll=False)

    start_in(g0, 0)

    def body(i, _):
        ga = g0 + 2 * i
        start_in(ga + 1, 1)
        wait_in(ga, 0)

        @pl.when(i > 0)
        def _():
            wait_out(0)

        shuffle(in_v.at[0], GW // 4, 0)
        start_out(ga, 0)

        @pl.when(2 * i + 2 < G_PER_W)
        def _():
            start_in(ga + 2, 0)

        wait_in(ga + 1, 1)

        @pl.when(i > 0)
        def _():
            wait_out(1)

        shuffle(in_v.at[1], GW // 4, 1)
        start_out(ga + 1, 1)
        return 0

    npairs = (G_PER_W - 1) // 2
    lax.fori_loop(0, npairs, body, 0, unroll=False)
    glast = g0 + G_PER_W - 1
    wait_in(glast, 0)
    wait_out(0)
    shuffle(in_v.at[0], GW // 4, 0)
    start_out(glast, 0)
    wait_out(1)
    wait_out(0)

    @pl.when(wid == NUM_WORKERS - 2)
    def _():
        start_in(NG_EVEN, 0)
        wait_in(NG_EVEN, 0)
        shuffle(in_v.at[0], GW // 4, 0)
        pltpu.sync_copy(
            out_v.at[0], r_hbm.at[pl.ds(NG_EVEN * (GW // 4), GW // 4), :]
        )

    @pl.when(wid == NUM_WORKERS - 1)
    def _():
        start = pl.multiple_of((NG_EVEN + 1) * GW + (wid - (NUM_WORKERS - 1)), 128)
        pltpu.sync_copy(tt_hbm.at[:, pl.ds(start, 128)], tail_v)
        shuffle(tail_v, TAIL_COLS // 4, 0)
        pltpu.sync_copy(
            out_v.at[0, pl.ds(0, TAIL_COLS // 4), :],
            r_hbm.at[pl.ds((NG_EVEN + 1) * (GW // 4), TAIL_COLS // 4), :],
        )

    def xrow_body(i, _):
        h = wid + i * NUM_WORKERS
        pltpu.sync_copy(xt_hbm.at[h], row_v)
        pltpu.sync_copy(row_v, xlin_hbm.at[pl.ds(h * BATCH, BATCH)])
        return 0

    xtrips = (HIST - 1 - wid) // NUM_WORKERS + 1
    lax.fori_loop(0, xtrips, xrow_body, 0, unroll=False)


@functools.partial(
    pl.kernel,
    mesh=_mesh,
    out_type=jax.ShapeDtypeStruct((HIST, 4, BATCH // 128, 8, 128), jnp.float32),
    scratch_types=[
        pltpu.VMEM((HIST, STRIPE), jnp.int32),
        pltpu.VMEM((2, STRIPE, D), jnp.float32),
        pltpu.VMEM((2, 4, 4, 8, 128), jnp.float32),
        pltpu.SemaphoreType.DMA,
        pltpu.SemaphoreType.DMA,
        pltpu.SemaphoreType.DMA,
        pltpu.SemaphoreType.DMA,
    ],
    compiler_params=pltpu.CompilerParams(
        needs_layout_passes=False, use_tc_tiling_on_sc=False
    ),
)
def _gather_all(x2_hbm, rl_hbm, out_hbm, i_v, rows_v, stage_v, sem0, sem1, os0, os1):
    wid = lax.axis_index("s") * 2 + lax.axis_index("c")
    c0 = wid * 4
    sems = (sem0, sem1)
    osems = (os0, os1)
    it = jax.lax.broadcasted_iota(jnp.int32, (16,), 0)

    pltpu.sync_copy(x2_hbm.at[:, pl.ds(c0 * 128, STRIPE)], i_v)

    def issue(h, buf):
        pltpu.async_copy(rl_hbm.at[i_v.at[h]], rows_v.at[buf], sems[buf])

    def wait_stage(h, buf):
        for ci in range(4):
            pltpu.make_async_copy(
                stage_v.at[buf, ci], out_hbm.at[h, :, c0 + ci, :, :], osems[buf]
            ).wait()

    def drain(h, buf):
        pltpu.make_async_copy(
            rl_hbm.at[i_v.at[h]], rows_v.at[buf], sems[buf]
        ).wait()

        @pl.when(h >= 2)
        def _():
            wait_stage(h, buf)

        for ci in range(4):

            def extract(m, _, _ci=ci):
                n0 = m * 16
                r_vec = (128 * _ci + n0) + it
                n_vec = n0 + it
                for d0 in range(D):
                    d_vec = d0 + it if d0 < 16 else (d0 + it) & (D - 1)
                    v = plsc.load_gather(rows_v.at[buf], [r_vec, d_vec])
                    plsc.store_scatter(
                        stage_v.at[buf, _ci],
                        [lax.shift_right_logical(d_vec, 3), d_vec & 7, n_vec],
                        v,
                    )
                return 0

            lax.fori_loop(0, 8, extract, 0, unroll=False)
        for ci in range(4):
            pltpu.async_copy(
                stage_v.at[buf, ci], out_hbm.at[h, :, c0 + ci, :, :], osems[buf]
            )

    issue(0, 0)

    def body(g, _):
        h = 2 * g
        issue(h + 1, 1)
        drain(h, 0)
        issue(h + 2, 0)
        drain(h + 1, 1)
        return 0

    lax.fori_loop(0, (HIST - 2) // 2, body, 0, unroll=False)
    issue(HIST - 1, 1)
    drain(HIST - 2, 0)
    drain(HIST - 1, 1)
    wait_stage(HIST - 2, 0)
    wait_stage(HIST - 1, 1)


def kernel(x, table):
    r, xlin = _reformat(x.T, table.T)
    rl = r.reshape(DATA_DIM, D)
    x2 = xlin.reshape(HIST, BATCH)
    out5 = _gather_all(x2, rl)
    out = out5.transpose(0, 1, 3, 2, 4).reshape(HIST, D, BATCH)
    return out.transpose(2, 0, 1)


# speedup vs baseline: 3.6156x; 1.0036x over previous
import functools

import jax
import jax.numpy as jnp
from jax import lax
from jax.experimental import pallas as pl
from jax.experimental.pallas import tpu as pltpu
from jax.experimental.pallas import tpu_sc as plsc

DATA_DIM = 1000000
D = 32
BATCH = 16384
HIST = 50
N = BATCH * HIST
NUM_WORKERS = 32
RROWS = DATA_DIM // 4
GW = 512
NG_EVEN = 1952
G_PER_W = NG_EVEN // NUM_WORKERS
TAIL_COLS = 64
STRIPE = 512

_mesh = plsc.VectorSubcoreMesh(core_axis_name="c", subcore_axis_name="s")


@functools.partial(
    pl.kernel,
    mesh=_mesh,
    out_type=(
        jax.ShapeDtypeStruct((RROWS, 128), jnp.float32),
        jax.ShapeDtypeStruct((N,), jnp.int32),
    ),
    scratch_types=[
        pltpu.VMEM((2, D, GW), jnp.float32),
        pltpu.VMEM((2, GW // 4, 128), jnp.float32),
        pltpu.VMEM((D, 128), jnp.float32),
        pltpu.VMEM((BATCH,), jnp.int32),
        pltpu.SemaphoreType.DMA,
        pltpu.SemaphoreType.DMA,
        pltpu.SemaphoreType.DMA,
        pltpu.SemaphoreType.DMA,
    ],
    compiler_params=pltpu.CompilerParams(needs_layout_passes=False),
)
def _reformat(
    xt_hbm, tt_hbm, r_hbm, xlin_hbm, in_v, out_v, tail_v, row_v, si0, si1, so0, so1
):
    wid = lax.axis_index("s") * 2 + lax.axis_index("c")
    g0 = wid * G_PER_W
    isems = (si0, si1)
    osems = (so0, so1)
    it = jax.lax.broadcasted_iota(jnp.int32, (16,), 0)
    c_e4 = lax.shift_right_logical(it, 2)
    c_k32 = (it & 3) * D

    def start_in(g, buf):
        pltpu.async_copy(tt_hbm.at[:, pl.ds(g * GW, GW)], in_v.at[buf], isems[buf])

    def wait_in(g, buf):
        pltpu.make_async_copy(
            tt_hbm.at[:, pl.ds(g * GW, GW)], in_v.at[buf], isems[buf]
        ).wait()

    def start_out(g, buf):
        pltpu.async_copy(
            out_v.at[buf], r_hbm.at[pl.ds(g * (GW // 4), GW // 4), :], osems[buf]
        )

    def wait_out(buf):
        pltpu.make_async_copy(
            out_v.at[buf], r_hbm.at[pl.ds(0, GW // 4), :], osems[buf]
        ).wait()

    def shuffle(src, nrows, obuf):
        def sdiag(m, _):
            e_vec = m * 16 + it
            er_vec = m * 4 + c_e4
            for d0 in range(D):
                d_vec = d0 + it if d0 < 16 else (d0 + it) & (D - 1)
                v = plsc.load_gather(src, [d_vec, e_vec])
                plsc.store_scatter(out_v.at[obuf], [er_vec, c_k32 + d_vec], v)
            return 0

        lax.fori_loop(0, nrows * 4 // 16, sdiag, 0, unroll=False)

    start_in(g0, 0)

    def body(i, _):
        ga = g0 + 2 * i
        start_in(ga + 1, 1)
        wait_in(ga, 0)

        @pl.when(i > 0)
        def _():
            wait_out(0)

        shuffle(in_v.at[0], GW // 4, 0)
        start_out(ga, 0)

        @pl.when(2 * i + 2 < G_PER_W)
        def _():
            start_in(ga + 2, 0)

        wait_in(ga + 1, 1)

        @pl.when(i > 0)
        def _():
            wait_out(1)

        shuffle(in_v.at[1], GW // 4, 1)
        start_out(ga + 1, 1)
        return 0

    npairs = (G_PER_W - 1) // 2
    lax.fori_loop(0, npairs, body, 0, unroll=False)
    glast = g0 + G_PER_W - 1
    wait_in(glast, 0)
    wait_out(0)
    shuffle(in_v.at[0], GW // 4, 0)
    start_out(glast, 0)
    wait_out(1)
    wait_out(0)

    @pl.when(wid == NUM_WORKERS - 2)
    def _():
        start_in(NG_EVEN, 0)
        wait_in(NG_EVEN, 0)
        shuffle(in_v.at[0], GW // 4, 0)
        pltpu.sync_copy(
            out_v.at[0], r_hbm.at[pl.ds(NG_EVEN * (GW // 4), GW // 4), :]
        )

    @pl.when(wid == NUM_WORKERS - 1)
    def _():
        start = pl.multiple_of((NG_EVEN + 1) * GW + (wid - (NUM_WORKERS - 1)), 128)
        pltpu.sync_copy(tt_hbm.at[:, pl.ds(start, 128)], tail_v)
        shuffle(tail_v, TAIL_COLS // 4, 0)
        pltpu.sync_copy(
            out_v.at[0, pl.ds(0, TAIL_COLS // 4), :],
            r_hbm.at[pl.ds((NG_EVEN + 1) * (GW // 4), TAIL_COLS // 4), :],
        )

    def xrow_body(i, _):
        h = wid + i * NUM_WORKERS
        pltpu.sync_copy(xt_hbm.at[h], row_v)
        pltpu.sync_copy(row_v, xlin_hbm.at[pl.ds(h * BATCH, BATCH)])
        return 0

    xtrips = (HIST - 1 - wid) // NUM_WORKERS + 1
    lax.fori_loop(0, xtrips, xrow_body, 0, unroll=False)


@functools.partial(
    pl.kernel,
    mesh=_mesh,
    out_type=jax.ShapeDtypeStruct((HIST, 4, BATCH // 128, 8, 128), jnp.float32),
    scratch_types=[
        pltpu.VMEM((HIST, STRIPE), jnp.int32),
        pltpu.VMEM((2, STRIPE, D), jnp.float32),
        pltpu.VMEM((2, 4, 4, 8, 128), jnp.float32),
        pltpu.SemaphoreType.DMA,
        pltpu.SemaphoreType.DMA,
        pltpu.SemaphoreType.DMA,
        pltpu.SemaphoreType.DMA,
    ],
    compiler_params=pltpu.CompilerParams(
        needs_layout_passes=False, use_tc_tiling_on_sc=False
    ),
)
def _gather_all(x2_hbm, rl_hbm, out_hbm, i_v, rows_v, stage_v, sem0, sem1, os0, os1):
    wid = lax.axis_index("s") * 2 + lax.axis_index("c")
    c0 = wid * 4
    sems = (sem0, sem1)
    osems = (os0, os1)
    it = jax.lax.broadcasted_iota(jnp.int32, (16,), 0)

    pltpu.sync_copy(x2_hbm.at[:, pl.ds(c0 * 128, STRIPE)], i_v)

    def issue(h, buf):
        pltpu.async_copy(rl_hbm.at[i_v.at[h]], rows_v.at[buf], sems[buf])

    def wait_stage(h, buf):
        pltpu.make_async_copy(
            stage_v.at[buf], out_hbm.at[h, :, pl.ds(c0, 4), :, :], osems[buf]
        ).wait()

    def drain(h, buf):
        pltpu.make_async_copy(
            rl_hbm.at[i_v.at[h]], rows_v.at[buf], sems[buf]
        ).wait()

        @pl.when(h >= 2)
        def _():
            wait_stage(h, buf)

        for ci in range(4):
            ci_vec = jnp.full((16,), ci, jnp.int32)

            def extract(m, _, _ci=ci, _cv=ci_vec):
                n0 = m * 16
                r_vec = (128 * _ci + n0) + it
                n_vec = n0 + it
                for d0 in range(D):
                    d_vec = d0 + it if d0 < 16 else (d0 + it) & (D - 1)
                    v = plsc.load_gather(rows_v.at[buf], [r_vec, d_vec])
                    plsc.store_scatter(
                        stage_v.at[buf],
                        [lax.shift_right_logical(d_vec, 3), _cv, d_vec & 7, n_vec],
                        v,
                    )
                return 0

            lax.fori_loop(0, 8, extract, 0, unroll=False)
        pltpu.async_copy(
            stage_v.at[buf], out_hbm.at[h, :, pl.ds(c0, 4), :, :], osems[buf]
        )

    issue(0, 0)

    def body(g, _):
        h = 2 * g
        issue(h + 1, 1)
        drain(h, 0)
        issue(h + 2, 0)
        drain(h + 1, 1)
        return 0

    lax.fori_loop(0, (HIST - 2) // 2, body, 0, unroll=False)
    issue(HIST - 1, 1)
    drain(HIST - 2, 0)
    drain(HIST - 1, 1)
    wait_stage(HIST - 2, 0)
    wait_stage(HIST - 1, 1)


def kernel(x, table):
    r, xlin = _reformat(x.T, table.T)
    rl = r.reshape(DATA_DIM, D)
    x2 = xlin.reshape(HIST, BATCH)
    out5 = _gather_all(x2, rl)
    out = out5.transpose(0, 1, 3, 2, 4).reshape(HIST, D, BATCH)
    return out.transpose(2, 0, 1)
